# Initial kernel scaffold; baseline (speedup 1.0000x reference)
import jax
import jax.numpy as jnp
from jax.experimental import pallas as pl


def kernel(user_embedding, item_embedding, ln_gamma, ln_beta, ui_attn_w, social_attn_w, ui_values, social_values, ui_edge_index, social_edge_index):
    raise NotImplementedError("write your pallas kernel here")


# speedup vs baseline: 4.0430x; 4.0430x over previous
import functools

import jax
import jax.numpy as jnp
from jax import lax
from jax.experimental import pallas as pl
from jax.experimental.pallas import tpu as pltpu
from jax.experimental.pallas import tpu_sc as plsc

N = 50000
D = 64
H = 32
E = 800000
NC = 2
NS = 16
CH = 640
CHUNKS = 80
EP = NS * CHUNKS * CH
RPT = N // NS
RSTG = 625

_f32 = jnp.float32
_i32 = jnp.int32

_GDN = lax.GatherDimensionNumbers(offset_dims=(), collapsed_slice_dims=(0,),
                                  start_index_map=(0,))


def _lane_bcast(v16, j):
    idx = jnp.full((16, 1), j, _i32)
    return lax.gather(v16, idx, _GDN, slice_sizes=(1,),
                      mode=lax.GatherScatterMode.PROMISE_IN_BOUNDS)


def _spmm_body(x2, base2, src2, dst, vals, out2, idx_v, dst_v, val_v, rows_v,
               acc, sem):
    c = lax.axis_index("c")
    s = lax.axis_index("s")
    base_row = c * N

    for k in range(RPT // RSTG):
        r0 = s * RPT + k * RSTG
        pltpu.sync_copy(base2.at[pl.ds(base_row + r0, RSTG)],
                        rows_v.at[pl.ds(0, RSTG)])
        pltpu.sync_copy(rows_v.at[pl.ds(0, RSTG)], acc.at[pl.ds(r0, RSTG)])
    plsc.subcore_barrier()

    def chunk(i, carry):
        e0 = s * (CHUNKS * CH) + i * CH
        pltpu.sync_copy(src2.at[pl.ds(c * EP + e0, CH)], idx_v)
        pltpu.sync_copy(dst.at[pl.ds(e0, CH)], dst_v)
        pltpu.sync_copy(vals.at[pl.ds(e0, CH)], val_v)
        pltpu.async_copy(x2.at[idx_v], rows_v, sem).wait()

        def grp(g, carry2):
            v16 = val_v[pl.ds(g * 16, 16)]
            for j in range(16):
                e = g * 16 + j
                bv = _lane_bcast(v16, j)
                rows_v[e, pl.ds(0, 16)] = rows_v[e, pl.ds(0, 16)] * bv
                rows_v[e, pl.ds(16, 16)] = rows_v[e, pl.ds(16, 16)] * bv
            return carry2

        lax.fori_loop(0, CH // 16, grp, 0)
        pltpu.sync_copy(rows_v, acc.at[dst_v], add=True)
        return carry

    lax.fori_loop(0, CHUNKS, chunk, 0)
    plsc.subcore_barrier()

    for k in range(RPT // RSTG):
        r0 = s * RPT + k * RSTG
        pltpu.sync_copy(acc.at[pl.ds(r0, RSTG)], rows_v.at[pl.ds(0, RSTG)])
        pltpu.sync_copy(rows_v.at[pl.ds(0, RSTG)],
                        out2.at[pl.ds(base_row + r0, RSTG)])


@functools.cache
def _spmm_kernel():
    return pl.kernel(
        _spmm_body,
        out_type=jax.ShapeDtypeStruct((2 * N, H), _f32),
        mesh=plsc.VectorSubcoreMesh(core_axis_name="c", subcore_axis_name="s"),
        scratch_types=[
            pltpu.VMEM((CH,), _i32),
            pltpu.VMEM((CH,), _i32),
            pltpu.VMEM((CH,), _f32),
            pltpu.VMEM((CH, H), _f32),
            pltpu.VMEM_SHARED((N, H), _f32),
            pltpu.SemaphoreType.DMA,
        ],
        compiler_params=pltpu.CompilerParams(use_tc_tiling_on_sc=False),
    )


def _spmm(x2, base2, src2, dst, vals):
    return _spmm_kernel()(x2, base2, src2, dst, vals)


def _ln_halves(i_ref, g, b):
    lo = i_ref[0]
    hi = i_ref[1]
    mu = (jnp.sum(lo, axis=1, keepdims=True)
          + jnp.sum(hi, axis=1, keepdims=True)) * (1.0 / D)
    dlo = lo - mu
    dhi = hi - mu
    var = (jnp.sum(dlo * dlo, axis=1, keepdims=True)
           + jnp.sum(dhi * dhi, axis=1, keepdims=True)) * (1.0 / D)
    inv = lax.rsqrt(var + 1e-5)
    ylo = dlo * inv * g[:H] + b[:H]
    yhi = dhi * inv * g[H:] + b[H:]
    return ylo, yhi


def _ln3_body(a_ref, b_ref, c_ref, g_ref, be_ref, ao_ref, bo_ref, co_ref):
    g = g_ref[0]
    be = be_ref[0]
    for i_ref, o_ref in ((a_ref, ao_ref), (b_ref, bo_ref), (c_ref, co_ref)):
        ylo, yhi = _ln_halves(i_ref, g, be)
        o_ref[0] = ylo
        o_ref[1] = yhi


def _ln3f_body(a_ref, b_ref, c_ref, g_ref, be_ref, ao_ref, co_ref, bf_ref):
    g = g_ref[0]
    be = be_ref[0]
    for i_ref, o_ref in ((a_ref, ao_ref), (c_ref, co_ref)):
        ylo, yhi = _ln_halves(i_ref, g, be)
        o_ref[0] = ylo
        o_ref[1] = yhi
    ylo, yhi = _ln_halves(b_ref, g, be)
    bf_ref[:, 0:H] = ylo
    bf_ref[:, H:D] = yhi


LB = 1000


def _halves_spec():
    return pl.BlockSpec((2, LB, H), lambda i: (0, i, 0))


def _vec_spec():
    return pl.BlockSpec((1, D), lambda i: (0, 0))


def _ln3(a, b, c, g2, b2):
    f = pl.pallas_call(
        _ln3_body,
        grid=(N // LB,),
        in_specs=[_halves_spec(), _halves_spec(), _halves_spec(),
                  _vec_spec(), _vec_spec()],
        out_specs=[_halves_spec(), _halves_spec(), _halves_spec()],
        out_shape=[jax.ShapeDtypeStruct((2, N, H), _f32)] * 3,
    )
    r = f(a.reshape(2, N, H), b.reshape(2, N, H), c.reshape(2, N, H), g2, b2)
    return tuple(x.reshape(2 * N, H) for x in r)


def _ln3_final(a, b, c, g2, b2):
    f = pl.pallas_call(
        _ln3f_body,
        grid=(N // LB,),
        in_specs=[_halves_spec(), _halves_spec(), _halves_spec(),
                  _vec_spec(), _vec_spec()],
        out_specs=[_halves_spec(), _halves_spec(),
                   pl.BlockSpec((LB, D), lambda i: (i, 0))],
        out_shape=[jax.ShapeDtypeStruct((2, N, H), _f32),
                   jax.ShapeDtypeStruct((2, N, H), _f32),
                   jax.ShapeDtypeStruct((N, D), _f32)],
    )
    ah, ch, bf = f(a.reshape(2, N, H), b.reshape(2, N, H),
                   c.reshape(2, N, H), g2, b2)
    return ah.reshape(2 * N, H), ch.reshape(2 * N, H), bf


def _attn_pair(stacks, w):
    scores = [jnp.sum(lo * w[:H], axis=1, keepdims=True)
              + jnp.sum(hi * w[H:], axis=1, keepdims=True)
              for lo, hi in stacks]
    m = jnp.maximum(jnp.maximum(scores[0], scores[1]), scores[2])
    es = [jnp.exp(sc - m) for sc in scores]
    denom = es[0] + es[1] + es[2]
    olo = sum(e * lo for e, (lo, _) in zip(es, stacks)) / denom
    ohi = sum(e * hi for e, (_, hi) in zip(es, stacks)) / denom
    return olo, ohi


def _attn_body(u0_ref, u1_ref, u2_ref, v1_ref, v2_ref, wu_ref, ws_ref, o_ref):
    wu = wu_ref[0]
    ws = ws_ref[0]
    u0 = (u0_ref[0], u0_ref[1])
    ui_lo, ui_hi = _attn_pair([u0, (u1_ref[0], u1_ref[1]),
                               (u2_ref[0], u2_ref[1])], wu)
    so_lo, so_hi = _attn_pair([u0, (v1_ref[0], v1_ref[1]),
                               (v2_ref[0], v2_ref[1])], ws)
    o_ref[:, 0:H] = ui_lo + so_lo
    o_ref[:, H:D] = ui_hi + so_hi


def _attn(u0h, u1h, u2h, us1h, us2h, wu2, ws2):
    f = pl.pallas_call(
        _attn_body,
        grid=(N // LB,),
        in_specs=[_halves_spec()] * 5 + [_vec_spec(), _vec_spec()],
        out_specs=pl.BlockSpec((LB, D), lambda i: (i, 0)),
        out_shape=jax.ShapeDtypeStruct((N, D), _f32),
    )
    return f(u0h.reshape(2, N, H), u1h.reshape(2, N, H), u2h.reshape(2, N, H),
             us1h.reshape(2, N, H), us2h.reshape(2, N, H), wu2, ws2)


def kernel(user_embedding, item_embedding, ln_gamma, ln_beta, ui_attn_w,
           social_attn_w, ui_values, social_values, ui_edge_index,
           social_edge_index):
    u_idx = ui_edge_index[0].astype(_i32)
    i_idx = ui_edge_index[1].astype(_i32)
    s_dst = social_edge_index[0].astype(_i32)
    s_src = social_edge_index[1].astype(_i32)

    pad = EP - E

    def prep_src2(src):
        sp = jnp.pad(src, (0, pad))
        return jnp.concatenate([sp, sp + N])

    def prep1(x):
        return jnp.pad(x, (0, pad))

    ui_gather_i = prep_src2(i_idx)
    ui_gather_u = prep_src2(u_idx)
    soc_gather = prep_src2(s_src)
    ui_dst_u = prep1(u_idx)
    ui_dst_i = prep1(i_idx)
    soc_dst = prep1(s_dst)
    ui_vals_p = prep1(ui_values)
    soc_vals_p = prep1(social_values)

    def halves(x):
        return jnp.concatenate([x[:, :H], x[:, H:]], axis=0)

    u0h = halves(user_embedding)
    i0h = halves(item_embedding)
    g2 = ln_gamma.reshape(1, D)
    b2 = ln_beta.reshape(1, D)
    wu2 = ui_attn_w.reshape(1, D)
    ws2 = social_attn_w.reshape(1, D)

    a1 = _spmm(i0h, u0h, ui_gather_i, ui_dst_u, ui_vals_p)
    a2 = _spmm(u0h, i0h, ui_gather_u, ui_dst_i, ui_vals_p)
    s1 = _spmm(u0h, u0h, soc_gather, soc_dst, soc_vals_p)
    u1h, i1h, us1h = _ln3(a1, a2, s1, g2, b2)

    a3 = _spmm(i1h, u1h, ui_gather_i, ui_dst_u, ui_vals_p)
    a4 = _spmm(u1h, i1h, ui_gather_u, ui_dst_i, ui_vals_p)
    s2 = _spmm(us1h, us1h, soc_gather, soc_dst, soc_vals_p)
    u2h, us2h, i2full = _ln3_final(a3, a4, s2, g2, b2)

    ufinal = _attn(u0h, u1h, u2h, us1h, us2h, wu2, ws2)
    return jnp.concatenate([ufinal, i2full], axis=0)
